# baseline (speedup 1.0000x reference)
import functools
import math

import jax
import jax.numpy as jnp
from jax.experimental import pallas as pl
from jax.experimental.pallas import tpu as pltpu

_CHUNK = 2048


def _flash_body(patch_ref, pos_ref, wl_ref, Wv_ref, bv_ref, Wo_ref,
                bo_ref, cls_ref, g1_ref, be1_ref, W1_ref, b1_ref, W2_ref,
                b2_ref, g2_ref, be2_ref, Wh_ref, bh_ref, out_ref,
                acc_ref, m_ref, d_ref, *, nc, h, dv, d_model):
    c = pl.program_id(1)

    @pl.when(c == 0)
    def _init():
        acc_ref[...] = jnp.zeros_like(acc_ref)
        m_ref[...] = jnp.full_like(m_ref, -jnp.inf)
        d_ref[...] = jnp.zeros_like(d_ref)

    h2 = 2 * h
    emb = patch_ref[0] + pos_ref[0]
    logitsT = jax.lax.dot_general(
        wl_ref[...], emb, (((1,), (1,)), ((), ())),
        preferred_element_type=jnp.float32)
    cmax = jnp.max(logitsT, axis=1, keepdims=True)
    m_old = m_ref[:, 0:1]
    m_new = jnp.maximum(m_old, cmax)
    alpha = jnp.exp(m_old - m_new)
    p = jnp.exp(logitsT - m_new)
    m_ref[...] = jnp.broadcast_to(m_new, m_ref.shape)
    d_new = d_ref[:, 0:1] * alpha + jnp.sum(p, axis=1, keepdims=True)
    d_ref[...] = jnp.broadcast_to(d_new, d_ref.shape)
    acc_ref[...] = acc_ref[...] * alpha + jax.lax.dot_general(
        p, emb, (((1,), (0,)), ((), ())), preferred_element_type=jnp.float32)

    @pl.when(c == nc - 1)
    def _epilogue():
        eps = 1e-5
        m_e = m_ref[0:h, 0:1]
        m_o = m_ref[h:h2, 0:1]
        mm = jnp.maximum(m_e, m_o)
        se = jnp.exp(m_e - mm)
        so = jnp.exp(m_o - mm)
        num = acc_ref[0:h, 0:d_model] * se + acc_ref[h:h2, d_model:] * so
        den = d_ref[0:h, 0:1] * se + d_ref[h:h2, 0:1] * so
        weighted = num / den
        full = jax.lax.dot_general(weighted, Wv_ref[...],
                                   (((1,), (0,)), ((), ())),
                                   preferred_element_type=jnp.float32)
        row = jax.lax.broadcasted_iota(jnp.int32, (h, h * dv), 0)
        colh = jax.lax.broadcasted_iota(jnp.int32, (h, h * dv), 1) // dv
        ctx = jnp.sum(jnp.where(row == colh, full, 0.0), axis=0,
                      keepdims=True) + bv_ref[...]
        out = jnp.dot(ctx, Wo_ref[...],
                      preferred_element_type=jnp.float32) + bo_ref[...]
        x = cls_ref[...] + out
        mu = jnp.mean(x, axis=1, keepdims=True)
        var = jnp.mean((x - mu) * (x - mu), axis=1, keepdims=True)
        x = (x - mu) / jnp.sqrt(var + eps) * g1_ref[...] + be1_ref[...]
        ff = jnp.maximum(
            jnp.dot(x, W1_ref[...], preferred_element_type=jnp.float32)
            + b1_ref[...], 0.0)
        ff = jnp.dot(ff, W2_ref[...],
                     preferred_element_type=jnp.float32) + b2_ref[...]
        y = x + ff
        mu2 = jnp.mean(y, axis=1, keepdims=True)
        var2 = jnp.mean((y - mu2) * (y - mu2), axis=1, keepdims=True)
        y = (y - mu2) / jnp.sqrt(var2 + eps) * g2_ref[...] + be2_ref[...]
        lg = jnp.dot(y, Wh_ref[...],
                     preferred_element_type=jnp.float32) + bh_ref[...]
        lg = lg - jnp.max(lg, axis=1, keepdims=True)
        e = jnp.exp(lg)
        out_ref[0] = e / jnp.sum(e, axis=1, keepdims=True)


def kernel(mem_patch, mem_pos, cls_token, W_q, b_q, W_k, b_k, W_v, b_v, W_o,
           b_o, ln1_g, ln1_b, W1, b1, W2, b2, ln2_g, ln2_b, W_head, b_head):
    Bb, Mm, Dd = mem_patch.shape
    n_class = W_head.shape[1]
    hdk = W_q.shape[1]
    dk = 16
    h = hdk // dk
    dv = W_v.shape[1] // h
    rows = Mm // 2
    nc = rows // _CHUNK

    q = (cls_token[0] @ W_q + b_q).reshape(h, dk) / math.sqrt(dk)
    wl = jnp.einsum('dhk,hk->dh', W_k.reshape(Dd, h, dk), q)
    wl2 = jnp.zeros((2 * h, 2 * Dd), jnp.float32)
    wl2 = wl2.at[:h, :Dd].set(wl.T).at[h:, Dd:].set(wl.T)

    pv = mem_patch.reshape(Bb, rows, 2 * Dd)
    qv = mem_pos.reshape(Bb, rows, 2 * Dd)

    row2 = lambda a: a.reshape(1, -1)
    full = lambda a: pl.BlockSpec(a.shape, lambda b, c: (0,) * a.ndim)

    weights = (wl2, W_v, row2(b_v), W_o, row2(b_o), cls_token[0],
               row2(ln1_g), row2(ln1_b), W1, row2(b1), W2, row2(b2),
               row2(ln2_g), row2(ln2_b), W_head, row2(b_head))

    grid = (Bb, nc)
    return pl.pallas_call(
        functools.partial(_flash_body, nc=nc, h=h, dv=dv, d_model=Dd),
        grid=grid,
        in_specs=[
            pl.BlockSpec((1, _CHUNK, 2 * Dd), lambda b, c: (b, c, 0)),
            pl.BlockSpec((1, _CHUNK, 2 * Dd), lambda b, c: (b, c, 0)),
        ] + [full(w) for w in weights],
        out_specs=pl.BlockSpec((1, 1, n_class), lambda b, c: (b, 0, 0)),
        out_shape=jax.ShapeDtypeStruct((Bb, 1, n_class), jnp.float32),
        scratch_shapes=[
            pltpu.VMEM((2 * h, 2 * Dd), jnp.float32),
            pltpu.VMEM((2 * h, 1), jnp.float32),
            pltpu.VMEM((2 * h, 1), jnp.float32),
        ],
    )(pv, qv, *weights)[:, 0, :]


# speedup vs baseline: 1.0988x; 1.0988x over previous
import functools
import math

import jax
import jax.numpy as jnp
from jax.experimental import pallas as pl
from jax.experimental.pallas import tpu as pltpu

_CHUNK = 2048
_HPAD = 8


def _flash_body(patch_ref, pos_ref, wl_ref, Wv_ref, bv_ref, Wo_ref,
                bo_ref, cls_ref, g1_ref, be1_ref, W1_ref, b1_ref, W2_ref,
                b2_ref, g2_ref, be2_ref, Wh_ref, bh_ref, out_ref,
                acc_ref, m_ref, d_ref, *, nc, h, dv, d_model):
    c = pl.program_id(1)

    @pl.when(c == 0)
    def _init():
        acc_ref[...] = jnp.zeros_like(acc_ref)
        m_ref[...] = jnp.full_like(m_ref, -jnp.inf)
        d_ref[...] = jnp.zeros_like(d_ref)

    emb = patch_ref[0] + pos_ref[0]
    logitsT = jax.lax.dot_general(
        wl_ref[...], emb, (((1,), (1,)), ((), ())),
        preferred_element_type=jnp.float32)
    cmax = jnp.max(logitsT, axis=1, keepdims=True)
    m_old = m_ref[:, 0:1]
    m_new = jnp.maximum(m_old, cmax)
    alpha = jnp.exp(m_old - m_new)
    p = jnp.exp(logitsT - m_new)
    m_ref[...] = jnp.broadcast_to(m_new, m_ref.shape)
    d_new = d_ref[:, 0:1] * alpha + jnp.sum(p, axis=1, keepdims=True)
    d_ref[...] = jnp.broadcast_to(d_new, d_ref.shape)
    acc_ref[...] = acc_ref[...] * alpha + jax.lax.dot_general(
        p, emb, (((1,), (0,)), ((), ())), preferred_element_type=jnp.float32)

    @pl.when(c == nc - 1)
    def _epilogue():
        eps = 1e-5
        weighted = acc_ref[0:h, :] / d_ref[0:h, 0:1]
        full = jax.lax.dot_general(weighted, Wv_ref[...],
                                   (((1,), (0,)), ((), ())),
                                   preferred_element_type=jnp.float32)
        row = jax.lax.broadcasted_iota(jnp.int32, (h, h * dv), 0)
        colh = jax.lax.broadcasted_iota(jnp.int32, (h, h * dv), 1) // dv
        ctx = jnp.sum(jnp.where(row == colh, full, 0.0), axis=0,
                      keepdims=True) + bv_ref[...]
        out = jnp.dot(ctx, Wo_ref[...],
                      preferred_element_type=jnp.float32) + bo_ref[...]
        x = cls_ref[...] + out
        mu = jnp.mean(x, axis=1, keepdims=True)
        var = jnp.mean((x - mu) * (x - mu), axis=1, keepdims=True)
        x = (x - mu) / jnp.sqrt(var + eps) * g1_ref[...] + be1_ref[...]
        ff = jnp.maximum(
            jnp.dot(x, W1_ref[...], preferred_element_type=jnp.float32)
            + b1_ref[...], 0.0)
        ff = jnp.dot(ff, W2_ref[...],
                     preferred_element_type=jnp.float32) + b2_ref[...]
        y = x + ff
        mu2 = jnp.mean(y, axis=1, keepdims=True)
        var2 = jnp.mean((y - mu2) * (y - mu2), axis=1, keepdims=True)
        y = (y - mu2) / jnp.sqrt(var2 + eps) * g2_ref[...] + be2_ref[...]
        lg = jnp.dot(y, Wh_ref[...],
                     preferred_element_type=jnp.float32) + bh_ref[...]
        lg = lg - jnp.max(lg, axis=1, keepdims=True)
        e = jnp.exp(lg)
        out_ref[0] = e / jnp.sum(e, axis=1, keepdims=True)


def kernel(mem_patch, mem_pos, cls_token, W_q, b_q, W_k, b_k, W_v, b_v, W_o,
           b_o, ln1_g, ln1_b, W1, b1, W2, b2, ln2_g, ln2_b, W_head, b_head):
    Bb, Mm, Dd = mem_patch.shape
    n_class = W_head.shape[1]
    hdk = W_q.shape[1]
    dk = 16
    h = hdk // dk
    dv = W_v.shape[1] // h
    nc = Mm // _CHUNK

    q = (cls_token[0] @ W_q + b_q).reshape(h, dk) / math.sqrt(dk)
    wl = jnp.einsum('dhk,hk->dh', W_k.reshape(Dd, h, dk), q)
    wl2 = jnp.zeros((_HPAD, Dd), jnp.float32).at[:h, :].set(wl.T)

    row2 = lambda a: a.reshape(1, -1)
    full = lambda a: pl.BlockSpec(a.shape, lambda b, c: (0,) * a.ndim)

    weights = (wl2, W_v, row2(b_v), W_o, row2(b_o), cls_token[0],
               row2(ln1_g), row2(ln1_b), W1, row2(b1), W2, row2(b2),
               row2(ln2_g), row2(ln2_b), W_head, row2(b_head))

    grid = (Bb, nc)
    return pl.pallas_call(
        functools.partial(_flash_body, nc=nc, h=h, dv=dv, d_model=Dd),
        grid=grid,
        in_specs=[
            pl.BlockSpec((1, _CHUNK, Dd), lambda b, c: (b, c, 0)),
            pl.BlockSpec((1, _CHUNK, Dd), lambda b, c: (b, c, 0)),
        ] + [full(w) for w in weights],
        out_specs=pl.BlockSpec((1, 1, n_class), lambda b, c: (b, 0, 0)),
        out_shape=jax.ShapeDtypeStruct((Bb, 1, n_class), jnp.float32),
        scratch_shapes=[
            pltpu.VMEM((_HPAD, Dd), jnp.float32),
            pltpu.VMEM((_HPAD, 1), jnp.float32),
            pltpu.VMEM((_HPAD, 1), jnp.float32),
        ],
    )(mem_patch, mem_pos, *weights)[:, 0, :]


# speedup vs baseline: 4.0298x; 3.6673x over previous
import functools
import math

import jax
import jax.numpy as jnp
from jax.experimental import pallas as pl
from jax.experimental.pallas import tpu as pltpu

_CHUNK = 4096
_HPAD = 8


def _flash_body(patch_ref, pos_ref, wl_ref, Wv_ref, bv_ref, Wo_ref,
                bo_ref, cls_ref, g1_ref, be1_ref, W1_ref, b1_ref, W2_ref,
                b2_ref, g2_ref, be2_ref, Wh_ref, bh_ref, out_ref,
                acc_ref, m_ref, d_ref, *, nc, h, dv):
    c = pl.program_id(1)

    @pl.when(c == 0)
    def _init():
        acc_ref[...] = jnp.zeros_like(acc_ref)
        m_ref[...] = jnp.full_like(m_ref, -jnp.inf)
        d_ref[...] = jnp.zeros_like(d_ref)

    embT = patch_ref[0] + pos_ref[0]
    logitsT = jax.lax.dot_general(
        wl_ref[...], embT, (((1,), (0,)), ((), ())),
        preferred_element_type=jnp.float32)
    cmax = jnp.max(logitsT, axis=1, keepdims=True)
    m_old = m_ref[:, 0:1]
    m_new = jnp.maximum(m_old, cmax)
    alpha = jnp.exp(m_old - m_new)
    p = jnp.exp(logitsT - m_new)
    m_ref[...] = jnp.broadcast_to(m_new, m_ref.shape)
    d_new = d_ref[:, 0:1] * alpha + jnp.sum(p, axis=1, keepdims=True)
    d_ref[...] = jnp.broadcast_to(d_new, d_ref.shape)
    acc_ref[...] = acc_ref[...] * alpha + jax.lax.dot_general(
        p, embT, (((1,), (1,)), ((), ())), preferred_element_type=jnp.float32)

    @pl.when(c == nc - 1)
    def _epilogue():
        eps = 1e-5
        weighted = acc_ref[0:h, :] / d_ref[0:h, 0:1]
        full = jax.lax.dot_general(weighted, Wv_ref[...],
                                   (((1,), (0,)), ((), ())),
                                   preferred_element_type=jnp.float32)
        row = jax.lax.broadcasted_iota(jnp.int32, (h, h * dv), 0)
        colh = jax.lax.broadcasted_iota(jnp.int32, (h, h * dv), 1) // dv
        ctx = jnp.sum(jnp.where(row == colh, full, 0.0), axis=0,
                      keepdims=True) + bv_ref[...]
        out = jnp.dot(ctx, Wo_ref[...],
                      preferred_element_type=jnp.float32) + bo_ref[...]
        x = cls_ref[...] + out
        mu = jnp.mean(x, axis=1, keepdims=True)
        var = jnp.mean((x - mu) * (x - mu), axis=1, keepdims=True)
        x = (x - mu) / jnp.sqrt(var + eps) * g1_ref[...] + be1_ref[...]
        ff = jnp.maximum(
            jnp.dot(x, W1_ref[...], preferred_element_type=jnp.float32)
            + b1_ref[...], 0.0)
        ff = jnp.dot(ff, W2_ref[...],
                     preferred_element_type=jnp.float32) + b2_ref[...]
        y = x + ff
        mu2 = jnp.mean(y, axis=1, keepdims=True)
        var2 = jnp.mean((y - mu2) * (y - mu2), axis=1, keepdims=True)
        y = (y - mu2) / jnp.sqrt(var2 + eps) * g2_ref[...] + be2_ref[...]
        lg = jnp.dot(y, Wh_ref[...],
                     preferred_element_type=jnp.float32) + bh_ref[...]
        lg = lg - jnp.max(lg, axis=1, keepdims=True)
        e = jnp.exp(lg)
        out_ref[0] = e / jnp.sum(e, axis=1, keepdims=True)


def kernel(mem_patch, mem_pos, cls_token, W_q, b_q, W_k, b_k, W_v, b_v, W_o,
           b_o, ln1_g, ln1_b, W1, b1, W2, b2, ln2_g, ln2_b, W_head, b_head):
    Bb, Mm, Dd = mem_patch.shape
    n_class = W_head.shape[1]
    hdk = W_q.shape[1]
    dk = 16
    h = hdk // dk
    dv = W_v.shape[1] // h
    nc = Mm // _CHUNK

    q = (cls_token[0] @ W_q + b_q).reshape(h, dk) / math.sqrt(dk)
    wl = jnp.einsum('dhk,hk->dh', W_k.reshape(Dd, h, dk), q)
    wl2 = jnp.zeros((_HPAD, Dd), jnp.float32).at[:h, :].set(wl.T)

    pT = jnp.swapaxes(mem_patch, 1, 2)
    qT = jnp.swapaxes(mem_pos, 1, 2)

    row2 = lambda a: a.reshape(1, -1)
    full = lambda a: pl.BlockSpec(a.shape, lambda b, c: (0,) * a.ndim)

    weights = (wl2, W_v, row2(b_v), W_o, row2(b_o), cls_token[0],
               row2(ln1_g), row2(ln1_b), W1, row2(b1), W2, row2(b2),
               row2(ln2_g), row2(ln2_b), W_head, row2(b_head))

    grid = (Bb, nc)
    return pl.pallas_call(
        functools.partial(_flash_body, nc=nc, h=h, dv=dv),
        grid=grid,
        in_specs=[
            pl.BlockSpec((1, Dd, _CHUNK), lambda b, c: (b, 0, c)),
            pl.BlockSpec((1, Dd, _CHUNK), lambda b, c: (b, 0, c)),
        ] + [full(w) for w in weights],
        out_specs=pl.BlockSpec((1, 1, n_class), lambda b, c: (b, 0, 0)),
        out_shape=jax.ShapeDtypeStruct((Bb, 1, n_class), jnp.float32),
        scratch_shapes=[
            pltpu.VMEM((_HPAD, Dd), jnp.float32),
            pltpu.VMEM((_HPAD, 1), jnp.float32),
            pltpu.VMEM((_HPAD, 1), jnp.float32),
        ],
    )(pT, qT, *weights)[:, 0, :]


# speedup vs baseline: 5.2753x; 1.3091x over previous
import functools
import math

import jax
import jax.numpy as jnp
from jax.experimental import pallas as pl
from jax.experimental.pallas import tpu as pltpu

_CHUNK = 8192
_HPAD = 8


def _flash_body(patch_ref, pos_ref, wl_ref, Wv_ref, bv_ref, Wo_ref,
                bo_ref, cls_ref, g1_ref, be1_ref, W1_ref, b1_ref, W2_ref,
                b2_ref, g2_ref, be2_ref, Wh_ref, bh_ref, out_ref,
                acc_ref, m_ref, d_ref, *, nc, h, dv):
    c = pl.program_id(1)

    @pl.when(c == 0)
    def _init():
        acc_ref[...] = jnp.zeros_like(acc_ref)
        m_ref[...] = jnp.full_like(m_ref, -jnp.inf)
        d_ref[...] = jnp.zeros_like(d_ref)

    embT = patch_ref[0] + pos_ref[0]
    logitsT = jax.lax.dot_general(
        wl_ref[...], embT, (((1,), (0,)), ((), ())),
        preferred_element_type=jnp.float32)
    cmax = jnp.max(logitsT, axis=1, keepdims=True)
    m_old = m_ref[:, 0:1]
    m_new = jnp.maximum(m_old, cmax)
    alpha = jnp.exp(m_old - m_new)
    p = jnp.exp(logitsT - m_new)
    m_ref[...] = jnp.broadcast_to(m_new, m_ref.shape)
    d_new = d_ref[:, 0:1] * alpha + jnp.sum(p, axis=1, keepdims=True)
    d_ref[...] = jnp.broadcast_to(d_new, d_ref.shape)
    acc_ref[...] = acc_ref[...] * alpha + jax.lax.dot_general(
        p, embT, (((1,), (1,)), ((), ())), preferred_element_type=jnp.float32)

    @pl.when(c == nc - 1)
    def _epilogue():
        eps = 1e-5
        weighted = acc_ref[0:h, :] / d_ref[0:h, 0:1]
        full = jax.lax.dot_general(weighted, Wv_ref[...],
                                   (((1,), (0,)), ((), ())),
                                   preferred_element_type=jnp.float32)
        row = jax.lax.broadcasted_iota(jnp.int32, (h, h * dv), 0)
        colh = jax.lax.broadcasted_iota(jnp.int32, (h, h * dv), 1) // dv
        ctx = jnp.sum(jnp.where(row == colh, full, 0.0), axis=0,
                      keepdims=True) + bv_ref[...]
        out = jnp.dot(ctx, Wo_ref[...],
                      preferred_element_type=jnp.float32) + bo_ref[...]
        x = cls_ref[...] + out
        mu = jnp.mean(x, axis=1, keepdims=True)
        var = jnp.mean((x - mu) * (x - mu), axis=1, keepdims=True)
        x = (x - mu) / jnp.sqrt(var + eps) * g1_ref[...] + be1_ref[...]
        ff = jnp.maximum(
            jnp.dot(x, W1_ref[...], preferred_element_type=jnp.float32)
            + b1_ref[...], 0.0)
        ff = jnp.dot(ff, W2_ref[...],
                     preferred_element_type=jnp.float32) + b2_ref[...]
        y = x + ff
        mu2 = jnp.mean(y, axis=1, keepdims=True)
        var2 = jnp.mean((y - mu2) * (y - mu2), axis=1, keepdims=True)
        y = (y - mu2) / jnp.sqrt(var2 + eps) * g2_ref[...] + be2_ref[...]
        lg = jnp.dot(y, Wh_ref[...],
                     preferred_element_type=jnp.float32) + bh_ref[...]
        lg = lg - jnp.max(lg, axis=1, keepdims=True)
        e = jnp.exp(lg)
        out_ref[0] = e / jnp.sum(e, axis=1, keepdims=True)


def kernel(mem_patch, mem_pos, cls_token, W_q, b_q, W_k, b_k, W_v, b_v, W_o,
           b_o, ln1_g, ln1_b, W1, b1, W2, b2, ln2_g, ln2_b, W_head, b_head):
    Bb, Mm, Dd = mem_patch.shape
    n_class = W_head.shape[1]
    hdk = W_q.shape[1]
    dk = 16
    h = hdk // dk
    dv = W_v.shape[1] // h
    nc = Mm // _CHUNK

    q = (cls_token[0] @ W_q + b_q).reshape(h, dk) / math.sqrt(dk)
    wl = jnp.einsum('dhk,hk->dh', W_k.reshape(Dd, h, dk), q)
    wl2 = jnp.zeros((_HPAD, Dd), jnp.float32).at[:h, :].set(wl.T)

    pT = jnp.swapaxes(mem_patch, 1, 2)
    qT = jnp.swapaxes(mem_pos, 1, 2)

    row2 = lambda a: a.reshape(1, -1)
    full = lambda a: pl.BlockSpec(a.shape, lambda b, c: (0,) * a.ndim)

    weights = (wl2, W_v, row2(b_v), W_o, row2(b_o), cls_token[0],
               row2(ln1_g), row2(ln1_b), W1, row2(b1), W2, row2(b2),
               row2(ln2_g), row2(ln2_b), W_head, row2(b_head))

    grid = (Bb, nc)
    return pl.pallas_call(
        functools.partial(_flash_body, nc=nc, h=h, dv=dv),
        grid=grid,
        in_specs=[
            pl.BlockSpec((1, Dd, _CHUNK), lambda b, c: (b, 0, c)),
            pl.BlockSpec((1, Dd, _CHUNK), lambda b, c: (b, 0, c)),
        ] + [full(w) for w in weights],
        out_specs=pl.BlockSpec((1, 1, n_class), lambda b, c: (b, 0, 0)),
        out_shape=jax.ShapeDtypeStruct((Bb, 1, n_class), jnp.float32),
        scratch_shapes=[
            pltpu.VMEM((_HPAD, Dd), jnp.float32),
            pltpu.VMEM((_HPAD, 1), jnp.float32),
            pltpu.VMEM((_HPAD, 1), jnp.float32),
        ],
    )(pT, qT, *weights)[:, 0, :]


# speedup vs baseline: 7.2499x; 1.3743x over previous
import functools
import math

import jax
import jax.numpy as jnp
from jax.experimental import pallas as pl
from jax.experimental.pallas import tpu as pltpu

_CHUNK = 16384
_HPAD = 8


def _flash_body(patch_ref, pos_ref, wl_ref, Wv_ref, bv_ref, Wo_ref,
                bo_ref, cls_ref, g1_ref, be1_ref, W1_ref, b1_ref, W2_ref,
                b2_ref, g2_ref, be2_ref, Wh_ref, bh_ref, out_ref,
                acc_ref, m_ref, d_ref, *, nc, h, dv):
    c = pl.program_id(1)

    @pl.when(c == 0)
    def _init():
        acc_ref[...] = jnp.zeros_like(acc_ref)
        m_ref[...] = jnp.full_like(m_ref, -jnp.inf)
        d_ref[...] = jnp.zeros_like(d_ref)

    embT = patch_ref[0] + pos_ref[0]
    logitsT = jax.lax.dot_general(
        wl_ref[...], embT, (((1,), (0,)), ((), ())),
        preferred_element_type=jnp.float32)
    cmax = jnp.max(logitsT, axis=1, keepdims=True)
    m_old = m_ref[:, 0:1]
    m_new = jnp.maximum(m_old, cmax)
    alpha = jnp.exp(m_old - m_new)
    p = jnp.exp(logitsT - m_new)
    m_ref[...] = jnp.broadcast_to(m_new, m_ref.shape)
    d_new = d_ref[:, 0:1] * alpha + jnp.sum(p, axis=1, keepdims=True)
    d_ref[...] = jnp.broadcast_to(d_new, d_ref.shape)
    acc_ref[...] = acc_ref[...] * alpha + jax.lax.dot_general(
        p, embT, (((1,), (1,)), ((), ())), preferred_element_type=jnp.float32)

    @pl.when(c == nc - 1)
    def _epilogue():
        eps = 1e-5
        weighted = acc_ref[0:h, :] / d_ref[0:h, 0:1]
        full = jax.lax.dot_general(weighted, Wv_ref[...],
                                   (((1,), (0,)), ((), ())),
                                   preferred_element_type=jnp.float32)
        row = jax.lax.broadcasted_iota(jnp.int32, (h, h * dv), 0)
        colh = jax.lax.broadcasted_iota(jnp.int32, (h, h * dv), 1) // dv
        ctx = jnp.sum(jnp.where(row == colh, full, 0.0), axis=0,
                      keepdims=True) + bv_ref[...]
        out = jnp.dot(ctx, Wo_ref[...],
                      preferred_element_type=jnp.float32) + bo_ref[...]
        x = cls_ref[...] + out
        mu = jnp.mean(x, axis=1, keepdims=True)
        var = jnp.mean((x - mu) * (x - mu), axis=1, keepdims=True)
        x = (x - mu) / jnp.sqrt(var + eps) * g1_ref[...] + be1_ref[...]
        ff = jnp.maximum(
            jnp.dot(x, W1_ref[...], preferred_element_type=jnp.float32)
            + b1_ref[...], 0.0)
        ff = jnp.dot(ff, W2_ref[...],
                     preferred_element_type=jnp.float32) + b2_ref[...]
        y = x + ff
        mu2 = jnp.mean(y, axis=1, keepdims=True)
        var2 = jnp.mean((y - mu2) * (y - mu2), axis=1, keepdims=True)
        y = (y - mu2) / jnp.sqrt(var2 + eps) * g2_ref[...] + be2_ref[...]
        lg = jnp.dot(y, Wh_ref[...],
                     preferred_element_type=jnp.float32) + bh_ref[...]
        lg = lg - jnp.max(lg, axis=1, keepdims=True)
        e = jnp.exp(lg)
        out_ref[0] = e / jnp.sum(e, axis=1, keepdims=True)


def kernel(mem_patch, mem_pos, cls_token, W_q, b_q, W_k, b_k, W_v, b_v, W_o,
           b_o, ln1_g, ln1_b, W1, b1, W2, b2, ln2_g, ln2_b, W_head, b_head):
    Bb, Mm, Dd = mem_patch.shape
    n_class = W_head.shape[1]
    hdk = W_q.shape[1]
    dk = 16
    h = hdk // dk
    dv = W_v.shape[1] // h
    nc = Mm // _CHUNK

    q = (cls_token[0] @ W_q + b_q).reshape(h, dk) / math.sqrt(dk)
    wl = jnp.einsum('dhk,hk->dh', W_k.reshape(Dd, h, dk), q)
    wl2 = jnp.zeros((_HPAD, Dd), jnp.float32).at[:h, :].set(wl.T)

    pT = jnp.swapaxes(mem_patch, 1, 2)
    qT = jnp.swapaxes(mem_pos, 1, 2)

    row2 = lambda a: a.reshape(1, -1)
    full = lambda a: pl.BlockSpec(a.shape, lambda b, c: (0,) * a.ndim)

    weights = (wl2, W_v, row2(b_v), W_o, row2(b_o), cls_token[0],
               row2(ln1_g), row2(ln1_b), W1, row2(b1), W2, row2(b2),
               row2(ln2_g), row2(ln2_b), W_head, row2(b_head))

    grid = (Bb, nc)
    return pl.pallas_call(
        functools.partial(_flash_body, nc=nc, h=h, dv=dv),
        grid=grid,
        in_specs=[
            pl.BlockSpec((1, Dd, _CHUNK), lambda b, c: (b, 0, c)),
            pl.BlockSpec((1, Dd, _CHUNK), lambda b, c: (b, 0, c)),
        ] + [full(w) for w in weights],
        out_specs=pl.BlockSpec((1, 1, n_class), lambda b, c: (b, 0, 0)),
        out_shape=jax.ShapeDtypeStruct((Bb, 1, n_class), jnp.float32),
        scratch_shapes=[
            pltpu.VMEM((_HPAD, Dd), jnp.float32),
            pltpu.VMEM((_HPAD, 1), jnp.float32),
            pltpu.VMEM((_HPAD, 1), jnp.float32),
        ],
    )(pT, qT, *weights)[:, 0, :]
